# Initial kernel scaffold; baseline (speedup 1.0000x reference)
import jax
import jax.numpy as jnp
from jax.experimental import pallas as pl


def kernel(src, table):
    raise NotImplementedError("write your pallas kernel here")


# speedup vs baseline: 7.7289x; 7.7289x over previous
import jax
import jax.numpy as jnp
from jax.experimental import pallas as pl

_PADDING_IDX = 0


def _body(t_ref, o_ref, *, block_rows):
    i = pl.program_id(0)
    x = t_ref[...]
    row = jax.lax.broadcasted_iota(jnp.int32, (block_rows, 1), 0) + i * block_rows
    x = jnp.where(row == _PADDING_IDX, 0.0, x)
    for b in range(o_ref.shape[1]):
        o_ref[:, b, :] = x


def kernel(src, table):
    seq_len, batch = src.shape
    max_len, hidden = table.shape
    block_rows = 256
    import functools
    out = pl.pallas_call(
        functools.partial(_body, block_rows=block_rows),
        grid=(seq_len // block_rows,),
        in_specs=[pl.BlockSpec((block_rows, hidden), lambda i: (i, 0))],
        out_specs=pl.BlockSpec((block_rows, batch, hidden), lambda i: (i, 0, 0)),
        out_shape=jax.ShapeDtypeStruct((seq_len, batch, hidden), table.dtype),
    )(table)
    return out
